# baseline (speedup 1.0000x reference)
import functools

import jax
import jax.numpy as jnp
from jax import lax
from jax.experimental import pallas as pl
from jax.experimental.pallas import tpu as pltpu
from jax.experimental.pallas import tpu_sc as plsc

ROWS = 8192
COLS = 1024
NUM_CORES = 2
NUM_SUBCORES = 16
NPAIR = NUM_SUBCORES // 2
ROWS_PER_SC = ROWS // NUM_CORES
CHUNK = 64
ROWS_PER_ROUND = NPAIR * CHUNK
NROUND = ROWS_PER_SC // ROWS_PER_ROUND

_MESH = plsc.VectorSubcoreMesh(
    core_axis_name="c", subcore_axis_name="s", num_cores=NUM_CORES
)


@functools.partial(
    pl.kernel,
    mesh=_MESH,
    out_type=jax.ShapeDtypeStruct((ROWS, COLS), jnp.float32),
    scratch_types=[
        pltpu.VMEM_SHARED((2, NPAIR, CHUNK, COLS), jnp.float32),
        pltpu.SemaphoreType.DMA,
    ],
)
def _copy_sc(pos_hbm, out_hbm, ring, sem):
    sid = lax.axis_index("s")
    base_sc = lax.axis_index("c") * ROWS_PER_SC
    is_reader = sid < NPAIR
    lane = lax.rem(sid, NPAIR)

    def row_base(r):
        return base_sc + (r * NPAIR) * CHUNK + lane * CHUNK

    for r in range(NROUND + 1):
        if r < NROUND:

            @pl.when(is_reader)
            def _():
                pltpu.make_async_copy(
                    pos_hbm.at[pl.ds(row_base(r), CHUNK)],
                    ring.at[r % 2, lane],
                    sem,
                ).start()

        if r >= 1:

            @pl.when(jnp.logical_not(is_reader))
            def _():
                pltpu.make_async_copy(
                    ring.at[(r - 1) % 2, lane],
                    out_hbm.at[pl.ds(row_base(r - 1), CHUNK)],
                    sem,
                ).start()

        if r < NROUND:

            @pl.when(is_reader)
            def _():
                pltpu.make_async_copy(
                    pos_hbm.at[pl.ds(row_base(r), CHUNK)],
                    ring.at[r % 2, lane],
                    sem,
                ).wait()

        if r >= 1:

            @pl.when(jnp.logical_not(is_reader))
            def _():
                pltpu.make_async_copy(
                    ring.at[(r - 1) % 2, lane],
                    out_hbm.at[pl.ds(row_base(r - 1), CHUNK)],
                    sem,
                ).wait()

        plsc.subcore_barrier()


def kernel(x, pos_emb):
    del x
    return _copy_sc(pos_emb)


# speedup vs baseline: 1.1168x; 1.1168x over previous
import functools

import jax
import jax.numpy as jnp
from jax import lax
from jax.experimental import pallas as pl
from jax.experimental.pallas import tpu as pltpu
from jax.experimental.pallas import tpu_sc as plsc

ROWS = 8192
COLS = 1024
SC_ROWS = 4096
NUM_CORES = 2
NUM_SUBCORES = 16
NUM_WORKERS = NUM_CORES * NUM_SUBCORES
ROWS_PER_WORKER = SC_ROWS // NUM_WORKERS
CHUNK = 32
NCHUNK = ROWS_PER_WORKER // CHUNK

_MESH = plsc.VectorSubcoreMesh(
    core_axis_name="c", subcore_axis_name="s", num_cores=NUM_CORES
)


@functools.partial(
    pl.kernel,
    mesh=_MESH,
    out_type=jax.ShapeDtypeStruct((ROWS, COLS), jnp.float32),
    scratch_types=[
        pltpu.VMEM((CHUNK, COLS), jnp.float32),
        pltpu.VMEM((CHUNK, COLS), jnp.float32),
        pltpu.VMEM((CHUNK, COLS), jnp.float32),
        pltpu.SemaphoreType.DMA,
        pltpu.SemaphoreType.DMA,
        pltpu.SemaphoreType.DMA,
        pltpu.SemaphoreType.DMA,
        pltpu.SemaphoreType.DMA,
        pltpu.SemaphoreType.DMA,
    ],
)
def _copy_sc(pos_hbm, out_hbm, buf0, buf1, buf2, rs0, rs1, rs2, ws0, ws1, ws2):
    wid = lax.axis_index("s") * NUM_CORES + lax.axis_index("c")
    base = wid * ROWS_PER_WORKER
    bufs = (buf0, buf1, buf2)
    rsems = (rs0, rs1, rs2)
    wsems = (ws0, ws1, ws2)
    nb = 3

    def read(i):
        return pltpu.make_async_copy(
            pos_hbm.at[pl.ds(base + i * CHUNK, CHUNK)], bufs[i % nb], rsems[i % nb]
        )

    def write(i):
        return pltpu.make_async_copy(
            bufs[i % nb], out_hbm.at[pl.ds(base + i * CHUNK, CHUNK)], wsems[i % nb]
        )

    read(0).start()
    read(1).start()
    for i in range(NCHUNK):
        read(i).wait()
        if i + 2 < NCHUNK:
            if i >= 1:
                write(i - 1).wait()
            read(i + 2).start()
        write(i).start()
    for j in range(max(0, NCHUNK - 2), NCHUNK):
        write(j).wait()
    if NCHUNK >= 3:
        write(NCHUNK - 3).wait()


def _copy_tc_body(src_ref, alias_ref, out_ref):
    del alias_ref
    out_ref[...] = src_ref[...]


def _copy_tc(pos_emb, sc_filled):
    block_rows = 1024
    grid = ((ROWS - SC_ROWS) // block_rows,)
    off = SC_ROWS // block_rows
    return pl.pallas_call(
        _copy_tc_body,
        grid=grid,
        in_specs=[
            pl.BlockSpec((block_rows, COLS), lambda i: (i + off, 0)),
            pl.BlockSpec(memory_space=pl.ANY),
        ],
        out_specs=pl.BlockSpec((block_rows, COLS), lambda i: (i + off, 0)),
        out_shape=jax.ShapeDtypeStruct((ROWS, COLS), jnp.float32),
        input_output_aliases={1: 0},
    )(pos_emb, sc_filled)


def kernel(x, pos_emb):
    del x
    head = _copy_sc(pos_emb)
    return _copy_tc(pos_emb, head)


# speedup vs baseline: 1.1697x; 1.0473x over previous
import functools

import jax
import jax.numpy as jnp
from jax import lax
from jax.experimental import pallas as pl
from jax.experimental.pallas import tpu as pltpu
from jax.experimental.pallas import tpu_sc as plsc

ROWS = 8192
COLS = 1024
NUM_CORES = 2
NUM_SUBCORES = 16
NUM_WORKERS = NUM_CORES * NUM_SUBCORES
ROWS_PER_WORKER = ROWS // NUM_WORKERS
CHUNK = 32
NCHUNK = ROWS_PER_WORKER // CHUNK

_MESH = plsc.VectorSubcoreMesh(
    core_axis_name="c", subcore_axis_name="s", num_cores=NUM_CORES
)


@functools.partial(
    pl.kernel,
    mesh=_MESH,
    out_type=jax.ShapeDtypeStruct((ROWS, COLS), jnp.float32),
    scratch_types=[
        pltpu.VMEM((CHUNK, COLS), jnp.float32),
        pltpu.VMEM((CHUNK, COLS), jnp.float32),
        pltpu.VMEM((CHUNK, COLS), jnp.float32),
        pltpu.SemaphoreType.DMA,
        pltpu.SemaphoreType.DMA,
        pltpu.SemaphoreType.DMA,
        pltpu.SemaphoreType.DMA,
        pltpu.SemaphoreType.DMA,
        pltpu.SemaphoreType.DMA,
    ],
)
def _copy_sc(pos_hbm, out_hbm, buf0, buf1, buf2, rs0, rs1, rs2, ws0, ws1, ws2):
    wid = lax.axis_index("s") * NUM_CORES + lax.axis_index("c")
    base = wid * ROWS_PER_WORKER
    bufs = (buf0, buf1, buf2)
    rsems = (rs0, rs1, rs2)
    wsems = (ws0, ws1, ws2)
    nb = 3

    def read(i):
        return pltpu.make_async_copy(
            pos_hbm.at[pl.ds(base + i * CHUNK, CHUNK)], bufs[i % nb], rsems[i % nb]
        )

    def write(i):
        return pltpu.make_async_copy(
            bufs[i % nb], out_hbm.at[pl.ds(base + i * CHUNK, CHUNK)], wsems[i % nb]
        )

    read(0).start()
    read(1).start()
    for i in range(NCHUNK):
        read(i).wait()
        if i + 2 < NCHUNK:
            if i >= 1:
                write(i - 1).wait()
            read(i + 2).start()
        write(i).start()
    for j in range(max(0, NCHUNK - 2), NCHUNK):
        write(j).wait()
    if NCHUNK >= 3:
        write(NCHUNK - 3).wait()


def kernel(x, pos_emb):
    del x
    return _copy_sc(pos_emb)
